# baseline (speedup 1.0000x reference)
import functools

import jax
import jax.numpy as jnp
from jax import lax
from jax.experimental import pallas as pl
from jax.experimental.pallas import tpu as pltpu
from jax.experimental.pallas import tpu_sc as plsc

_HARD_THRE_P = 0.5
_RANDOM_THRE_P = 0.1


def _res_body(x_ref, y_ref, out_ref):
    cc = pl.program_id(1)
    partial = jnp.sum(jnp.abs(x_ref[0] - y_ref[0]), axis=0)

    @pl.when(cc == 0)
    def _():
        out_ref[0] = partial

    @pl.when(cc != 0)
    def _():
        out_ref[0] += partial


def _compute_random_mask_packed(b, h, w):
    random_thre_ind = int(_RANDOM_THRE_P * w * h)
    base = jnp.concatenate([
        jnp.ones((random_thre_ind,), dtype=jnp.float32),
        jnp.zeros((h * w - random_thre_ind,), dtype=jnp.float32),
    ])
    keys = jax.random.split(jax.random.key(42), b)
    rm = jax.vmap(lambda kk: jax.random.permutation(kk, base))(keys)
    rm_u = rm.reshape(b, h * w // 32, 32).astype(jnp.uint32)
    weights = jnp.left_shift(jnp.uint32(1), jnp.arange(32, dtype=jnp.uint32))
    packed = jnp.sum(rm_u * weights, axis=-1, dtype=jnp.uint32)
    return jax.lax.bitcast_convert_type(packed, jnp.int32)


_PACKED_CACHE = {}


def _random_mask_packed(b, h, w):
    key = (b, h, w)
    if key not in _PACKED_CACHE:
        val = _compute_random_mask_packed(b, h, w)
        if isinstance(val, jax.core.Tracer):
            return val
        _PACKED_CACHE[key] = val
    return _PACKED_CACHE[key]


try:
    _random_mask_packed(4, 384, 384)
except Exception:
    pass

_LEVELS = ((23, 8), (15, 8), (7, 8), (0, 7))


def _sc_select_body(res_ref, bits_ref, pk_ref, out_ref, data_v, bits_v, pk_v,
                    hist_v, hist16_v, gath_v, acc_v, gathf_v, shist_s,
                    sred_s, *, nbatch, n, k):
    c = lax.axis_index("c")
    s = lax.axis_index("s")
    per = n // 16
    nv = per // 16
    pw = per // 32
    pwp = 128 * ((pw + 127) // 128)
    iota16 = jnp.arange(16, dtype=jnp.int32)
    ones16 = jnp.ones((16,), jnp.int32)

    for bi in range(nbatch // 2):
        b = 2 * c + bi

        pltpu.sync_copy(res_ref.at[b, pl.ds(s * per, per)], data_v.at[bi])
        pltpu.sync_copy(bits_ref.at[b, pl.ds(s * per, per)], bits_v.at[bi])
        pltpu.sync_copy(pk_ref.at[b, s], pk_v.at[pl.ds(bi * pwp, pwp)])

        kp1 = jnp.int32(k + 1)
        prefix = jnp.int32(0)

        for (shift, width) in _LEVELS:
            nbins = 1 << width
            hi_sh = shift + width

            def zbody(j, carry):
                hist16_v[pl.ds(j * 16, 16)] = jnp.zeros((16,), jnp.int32)
                return carry

            lax.fori_loop(0, 256, zbody, jnp.int32(0))

            prefix_hi = lax.shift_right_logical(prefix, jnp.int32(hi_sh))
            lane_off = iota16 * 256

            def hbody(i, carry):
                vb = bits_v[bi, pl.ds(i * 16, 16)]
                dig = lax.shift_right_logical(vb, jnp.int32(shift)) & (nbins - 1)
                act = lax.shift_right_logical(vb, jnp.int32(hi_sh)) == prefix_hi
                plsc.addupdate_scatter(hist16_v, [lane_off + dig], ones16,
                                       mask=act)
                return carry

            lax.fori_loop(0, nv, hbody, jnp.int32(0))

            for j in range(nbins // 16):
                hist_v[pl.ds(j * 16, 16)] = jnp.zeros((16,), jnp.int32)

            def fbody(r, carry):
                def finner(j, carry2):
                    hist_v[pl.ds(j * 16, 16)] += hist16_v[
                        pl.ds(r * 256 + j * 16, 16)]
                    return carry2
                return lax.fori_loop(0, nbins // 16, finner, carry)

            lax.fori_loop(0, 16, fbody, jnp.int32(0))

            pltpu.sync_copy(hist_v.at[pl.ds(0, nbins)],
                            shist_s.at[pl.ds(bi * 4096 + s * 256, nbins)])
            plsc.subcore_barrier()
            pltpu.sync_copy(shist_s.at[pl.ds(bi * 4096, 4096)], gath_v)
            plsc.subcore_barrier()

            def cbody(r, carry):
                def cinner(j, carry2):
                    hist_v[pl.ds(j * 16, 16)] += gath_v[
                        pl.ds(r * 256 + j * 16, 16)]
                    return carry2
                return lax.fori_loop(0, nbins // 16, cinner, carry)

            for j in range(nbins // 16):
                hist_v[pl.ds(j * 16, 16)] = jnp.zeros((16,), jnp.int32)
            lax.fori_loop(0, 16, cbody, jnp.int32(0))

            def gbody(gi, carry):
                done, dd, cntgt, running = carry
                g = (nbins // 16 - 1) - gi
                hv = hist_v[pl.ds(g * 16, 16)]
                ssum = jnp.sum(hv)
                suf = lax.rev(plsc.cumsum(lax.rev(hv, (0,))), (0,)) + running
                ok = (suf >= kp1).astype(jnp.int32)
                nok = jnp.sum(ok)
                hit = jnp.logical_and(done == 0, nok > 0)
                lmax = nok - 1
                sel = iota16 == lmax
                h_d = jnp.sum(jnp.where(sel, hv, 0))
                s_d = jnp.sum(jnp.where(sel, suf, 0))
                done2 = jnp.where(hit, jnp.int32(1), done)
                dd2 = jnp.where(hit, g * 16 + lmax, dd)
                cnt2 = jnp.where(hit, s_d - h_d, cntgt)
                run2 = jnp.where(done == 0, running + ssum, running)
                return done2, dd2, cnt2, run2

            _, dsel, cntgt, _ = lax.fori_loop(
                0, nbins // 16, gbody,
                (jnp.int32(0), jnp.int32(0), jnp.int32(0), jnp.int32(0)))

            prefix = prefix | lax.shift_left(dsel, jnp.int32(shift))
            kp1 = kp1 - cntgt

        thre_v = jnp.full((16,), prefix, jnp.int32)

        def sbody(i, acc):
            v = data_v[bi, pl.ds(i * 16, 16)]
            vb = bits_v[bi, pl.ds(i * 16, 16)]
            wv = plsc.load_gather(
                pk_v, [jnp.full((16,), bi * pwp + i // 2, jnp.int32)])
            sh = (i % 2) * 16 + iota16
            bitv = lax.shift_right_logical(wv, sh) & 1
            m = jnp.logical_or(vb > thre_v, bitv != 0)
            return acc + jnp.where(m, v, jnp.float32(0.0))

        acc = lax.fori_loop(0, nv, sbody, jnp.zeros((16,), jnp.float32))

        acc_v[...] = acc
        pltpu.sync_copy(acc_v, sred_s.at[pl.ds(bi * 256 + s * 16, 16)])
        plsc.subcore_barrier()

        @pl.when(s == 0)
        def _():
            pltpu.sync_copy(sred_s.at[pl.ds(bi * 256, 256)], gathf_v)
            tot = jnp.zeros((16,), jnp.float32)
            for r in range(16):
                tot = tot + gathf_v[pl.ds(r * 16, 16)]
            acc_v[...] = tot
            pltpu.sync_copy(acc_v, out_ref.at[b])

        plsc.subcore_barrier()


def kernel(x, y):
    b, c, h, w = x.shape
    cb = 8
    assert c % cb == 0 and w % 32 == 0

    res = pl.pallas_call(
        _res_body,
        grid=(b, c // cb),
        in_specs=[
            pl.BlockSpec((1, cb, h, w), lambda i, j: (i, j, 0, 0)),
            pl.BlockSpec((1, cb, h, w), lambda i, j: (i, j, 0, 0)),
        ],
        out_specs=pl.BlockSpec((1, h, w), lambda i, j: (i, 0, 0)),
        out_shape=jax.ShapeDtypeStruct((b, h, w), jnp.float32),
        compiler_params=pltpu.CompilerParams(
            dimension_semantics=("arbitrary", "arbitrary"),
        ),
    )(x, y)

    n = h * w
    res2 = res.reshape(b, n)
    bits2 = jax.lax.bitcast_convert_type(res2, jnp.int32)
    pw0 = n // (32 * 16)
    pwp0 = 128 * ((pw0 + 127) // 128)
    packed = _random_mask_packed(b, h, w).reshape(b, 16, pw0)
    packed = jnp.pad(packed, ((0, 0), (0, 0), (0, pwp0 - pw0)))
    k = int(_HARD_THRE_P * n)
    per = n // 16

    sc_fn = functools.partial(
        pl.kernel,
        mesh=plsc.VectorSubcoreMesh(core_axis_name="c", subcore_axis_name="s"),
        out_type=jax.ShapeDtypeStruct((b, 16), jnp.float32),
        compiler_params=pltpu.CompilerParams(needs_layout_passes=False),
        scratch_types=[
            pltpu.VMEM((b // 2, per), jnp.float32),
            pltpu.VMEM((b // 2, per), jnp.int32),
            pltpu.VMEM((b // 2 * 128 * ((per // 32 + 127) // 128),), jnp.int32),
            pltpu.VMEM((256,), jnp.int32),
            pltpu.VMEM((4096,), jnp.int32),
            pltpu.VMEM((4096,), jnp.int32),
            pltpu.VMEM((16,), jnp.float32),
            pltpu.VMEM((256,), jnp.float32),
            pltpu.VMEM_SHARED((b // 2 * 4096,), jnp.int32),
            pltpu.VMEM_SHARED((b // 2 * 256,), jnp.float32),
        ],
    )(functools.partial(_sc_select_body, nbatch=b, n=n, k=k))

    sums = sc_fn(res2, bits2, packed)
    return jnp.sum(sums) / (b * c * h * w)


# speedup vs baseline: 1.6358x; 1.6358x over previous
import functools

import jax
import jax.numpy as jnp
from jax.experimental import pallas as pl
from jax.experimental.pallas import tpu as pltpu

_HARD_THRE_P = 0.5
_RANDOM_THRE_P = 0.1


def _res_body(x_ref, y_ref, out_ref):
    cc = pl.program_id(1)
    partial = jnp.sum(jnp.abs(x_ref[0] - y_ref[0]), axis=0)

    @pl.when(cc == 0)
    def _():
        out_ref[0] = partial

    @pl.when(cc != 0)
    def _():
        out_ref[0] += partial


def _select_body(res_ref, rmask_ref, out_ref, *, k):
    res = res_ref[...]
    bits = jax.lax.bitcast_convert_type(res, jnp.int32)
    kv = jnp.full((res.shape[0], 1, 1), k + 1, dtype=jnp.int32)

    def body(i, p):
        t = p | jnp.left_shift(jnp.int32(1), 30 - i)
        part = jnp.sum((bits >= t).astype(jnp.int32), axis=1, keepdims=True)
        cnt = jnp.sum(part, axis=2, keepdims=True)
        return jnp.where(cnt >= kv, t, p)

    vbits = jax.lax.fori_loop(
        0, 31, body, jnp.zeros((res.shape[0], 1, 1), jnp.int32)
    )
    thre = jax.lax.bitcast_convert_type(vbits, jnp.float32)

    mask = jnp.logical_or(res > thre, rmask_ref[...] != 0)
    psum = jnp.sum(jnp.where(mask, res, 0.0), axis=1, keepdims=True)
    out_ref[0, 0] = jnp.sum(psum)


def _compute_random_mask_packed(b, h, w):
    random_thre_ind = int(_RANDOM_THRE_P * w * h)
    base = jnp.concatenate([
        jnp.ones((random_thre_ind,), dtype=jnp.float32),
        jnp.zeros((h * w - random_thre_ind,), dtype=jnp.float32),
    ])
    keys = jax.random.split(jax.random.key(42), b)
    rm = jax.vmap(lambda kk: jax.random.permutation(kk, base))(keys)
    rm_u = rm.reshape(b, h, w // 32, 32).astype(jnp.uint32)
    weights = jnp.left_shift(
        jnp.uint32(1), jnp.arange(32, dtype=jnp.uint32)
    )
    packed = jnp.sum(rm_u * weights, axis=-1, dtype=jnp.uint32)
    return jax.lax.bitcast_convert_type(packed, jnp.int32)


_PACKED_CACHE = {}


def _random_mask_packed(b, h, w):
    key = (b, h, w)
    if key not in _PACKED_CACHE:
        val = _compute_random_mask_packed(b, h, w)
        if isinstance(val, jax.core.Tracer):
            return val
        _PACKED_CACHE[key] = val
    return _PACKED_CACHE[key]


try:
    _random_mask_packed(4, 384, 384)
except Exception:
    pass


def kernel(x, y):
    b, c, h, w = x.shape
    cb = 8
    assert c % cb == 0 and w % 32 == 0

    res = pl.pallas_call(
        _res_body,
        grid=(b, c // cb),
        in_specs=[
            pl.BlockSpec((1, cb, h, w), lambda i, j: (i, j, 0, 0)),
            pl.BlockSpec((1, cb, h, w), lambda i, j: (i, j, 0, 0)),
        ],
        out_specs=pl.BlockSpec((1, h, w), lambda i, j: (i, 0, 0)),
        out_shape=jax.ShapeDtypeStruct((b, h, w), jnp.float32),
        compiler_params=pltpu.CompilerParams(
            dimension_semantics=("arbitrary", "arbitrary"),
        ),
    )(x, y)

    packed = _random_mask_packed(b, h, w)
    words = jnp.repeat(packed, 32, axis=2)
    shifts = (jnp.arange(w, dtype=jnp.int32) % 32)[None, None, :]
    rmask = jax.lax.shift_right_logical(words, shifts) & 1

    k = int(_HARD_THRE_P * w * h)

    total = pl.pallas_call(
        functools.partial(_select_body, k=k),
        in_specs=[
            pl.BlockSpec((b, h, w), lambda: (0, 0, 0)),
            pl.BlockSpec((b, h, w), lambda: (0, 0, 0)),
        ],
        out_specs=pl.BlockSpec(memory_space=pltpu.SMEM),
        out_shape=jax.ShapeDtypeStruct((1, 1), jnp.float32),
    )(res, rmask)

    return total[0, 0] / (b * c * h * w)
